# baseline (speedup 1.0000x reference)
import functools

import jax
import jax.numpy as jnp
from jax import lax
from jax.experimental import pallas as pl
from jax.experimental.pallas import tpu as pltpu
from jax.experimental.pallas import tpu_sc as plsc

_DIMS = ([16, 32, 64] * 8) + [16, 32]
_VOCAB = 100000
_VBLK = 512
_NVBLK = (_VOCAB + _VBLK - 1) // _VBLK
_VPAD = _NVBLK * _VBLK
_BATCH = 16384
_NC = 2
_NS = 16
_NW = _NC * _NS
_BPW = _BATCH // _NW
_CHUNK = 128
_NCHUNK = _BPW // _CHUNK


def _repack(wt, n_tab, d):
    blk_rows = _VBLK * d // 128
    pack = 128 // d

    def body(in_ref, o_ref):
        blk = in_ref[0]
        r = blk.reshape(d, blk_rows, pack)
        o_ref[...] = r.transpose(1, 2, 0).reshape(blk_rows, 128)

    return pl.pallas_call(
        body,
        grid=(n_tab, _NVBLK),
        in_specs=[pl.BlockSpec((1, d, _VBLK), lambda t, c: (t, 0, c))],
        out_specs=pl.BlockSpec((blk_rows, 128), lambda t, c: (t * _NVBLK + c, 0)),
        out_shape=jax.ShapeDtypeStruct((n_tab * _NVBLK * blk_rows, 128), jnp.float32),
    )(wt)


@functools.partial(jax.jit, static_argnums=())
def kernel(x, W16, W32, W64):
    p16 = _repack(W16.swapaxes(1, 2), 9, 16)
    p32 = _repack(W32.swapaxes(1, 2), 9, 32)
    p64 = _repack(W64.swapaxes(1, 2), 8, 64)
    s16 = p16.reshape(9 * _VPAD, 16)
    s32 = p32.reshape(9 * _VPAD, 32)
    s64 = p64.reshape(8 * _VPAD, 64)

    counters = {16: 0, 32: 0, 64: 0}
    offs = []
    for d in _DIMS:
        offs.append(counters[d] * _VPAD)
        counters[d] += 1
    offs = jnp.asarray(offs, dtype=jnp.int32)
    xw = (x.T + offs[:, None]).reshape(26, _NW, _NCHUNK, _CHUNK)
    xw = xw.transpose(1, 0, 2, 3).reshape(_NW, 26 * _NCHUNK, _CHUNK)

    mesh = plsc.VectorSubcoreMesh(core_axis_name="c", subcore_axis_name="s")
    out_type = tuple(
        jax.ShapeDtypeStruct((_BATCH, d), jnp.float32) for d in _DIMS
    )

    @functools.partial(
        pl.kernel,
        mesh=mesh,
        out_type=out_type,
        compiler_params=pltpu.CompilerParams(use_tc_tiling_on_sc=False),
        scratch_types=[
            pltpu.VMEM((26 * _NCHUNK, _CHUNK), jnp.int32),
            pltpu.VMEM((_BPW, 16), jnp.float32),
            pltpu.VMEM((_BPW, 32), jnp.float32),
            pltpu.VMEM((_BPW, 64), jnp.float32),
            pltpu.SemaphoreType.DMA,
            pltpu.SemaphoreType.DMA,
        ],
    )
    def run(xw_hbm, t16, t32, t64, *rest):
        outs = rest[:26]
        idx_v, r16, r32, r64, gsem, wsem = rest[26:]
        tabs = {16: t16, 32: t32, 64: t64}
        bufs = {16: r16, 32: r32, 64: r64}
        wid = lax.axis_index("s") * _NC + lax.axis_index("c")
        base = wid * _BPW
        pltpu.sync_copy(xw_hbm.at[wid], idx_v)

        def fire(f):
            d = _DIMS[f]
            return [
                pltpu.async_copy(
                    tabs[d].at[idx_v.at[f * _NCHUNK + j]],
                    bufs[d].at[pl.ds(j * _CHUNK, _CHUNK)],
                    gsem,
                )
                for j in range(_NCHUNK)
            ]

        writeback = {16: None, 32: None, 64: None}
        inflight = fire(0)
        for f in range(26):
            d = _DIMS[f]
            if f + 1 < 26:
                dn = _DIMS[f + 1]
                if writeback[dn] is not None:
                    writeback[dn].wait()
                    writeback[dn] = None
                nxt = fire(f + 1)
            for c in inflight:
                c.wait()
            writeback[d] = pltpu.async_copy(
                bufs[d], outs[f].at[pl.ds(base, _BPW)], wsem
            )
            if f + 1 < 26:
                inflight = nxt
        for d in (16, 32, 64):
            if writeback[d] is not None:
                writeback[d].wait()

    return run(xw, s16, s32, s64)


# speedup vs baseline: 7.9663x; 7.9663x over previous
import functools

import jax
import jax.numpy as jnp
from jax import lax
from jax.experimental import pallas as pl
from jax.experimental.pallas import tpu as pltpu
from jax.experimental.pallas import tpu_sc as plsc

_DIMS = ([16, 32, 64] * 8) + [16, 32]
_VOCAB = 100000
_BATCH = 16384
_NC = 2
_NS = 16
_NW = _NC * _NS
_BPW = _BATCH // _NW
_CHUNK = 128
_NCHUNK = _BPW // _CHUNK

_GROUPS = {
    16: [f for f in range(26) if _DIMS[f] == 16],
    32: [f for f in range(26) if _DIMS[f] == 32],
    64: [f for f in range(26) if _DIMS[f] == 64],
}


def _gather_group(d, fields, xw, stack):
    nf = len(fields)
    mesh = plsc.VectorSubcoreMesh(core_axis_name="c", subcore_axis_name="s")
    out_type = tuple(
        jax.ShapeDtypeStruct((_BATCH, d), jnp.float32) for _ in fields
    )

    @functools.partial(
        pl.kernel,
        mesh=mesh,
        out_type=out_type,
        compiler_params=pltpu.CompilerParams(use_tc_tiling_on_sc=False),
        scratch_types=[
            pltpu.VMEM((nf * _NCHUNK, _CHUNK), jnp.int32),
            pltpu.VMEM((_BPW, d), jnp.float32),
            pltpu.VMEM((_BPW, d), jnp.float32),
            pltpu.SemaphoreType.DMA,
            pltpu.SemaphoreType.DMA,
        ],
    )
    def run(xw_hbm, tab, *rest):
        outs = rest[:nf]
        idx_v, buf0, buf1, gsem, wsem = rest[nf:]
        bufs = (buf0, buf1)
        wid = lax.axis_index("s") * _NC + lax.axis_index("c")
        base = wid * _BPW
        pltpu.sync_copy(xw_hbm.at[wid], idx_v)

        def fire(i):
            buf = bufs[i % 2]
            return [
                pltpu.async_copy(
                    tab.at[idx_v.at[i * _NCHUNK + j]],
                    buf.at[pl.ds(j * _CHUNK, _CHUNK)],
                    gsem,
                )
                for j in range(_NCHUNK)
            ]

        writeback = [None, None]
        inflight = fire(0)
        for i in range(nf):
            if i + 1 < nf:
                if writeback[(i + 1) % 2] is not None:
                    writeback[(i + 1) % 2].wait()
                    writeback[(i + 1) % 2] = None
                nxt = fire(i + 1)
            for c in inflight:
                c.wait()
            writeback[i % 2] = pltpu.async_copy(
                bufs[i % 2], outs[i].at[pl.ds(base, _BPW)], wsem
            )
            if i + 1 < nf:
                inflight = nxt
        for wb in writeback:
            if wb is not None:
                wb.wait()

    return run(xw, stack)


@functools.partial(jax.jit, static_argnums=())
def kernel(x, W16, W32, W64):
    stacks = {
        16: W16.reshape(9 * _VOCAB, 16),
        32: W32.reshape(9 * _VOCAB, 32),
        64: W64.reshape(8 * _VOCAB, 64),
    }
    results = [None] * 26
    xt = x.T
    for d, fields in _GROUPS.items():
        nf = len(fields)
        rows = [xt[f] + jnp.int32(i * _VOCAB) for i, f in enumerate(fields)]
        xg = jnp.stack(rows).reshape(nf, _NW, _NCHUNK, _CHUNK)
        xg = xg.transpose(1, 0, 2, 3).reshape(_NW, nf * _NCHUNK, _CHUNK)
        outs = _gather_group(d, fields, xg, stacks[d])
        for f, o in zip(fields, outs):
            results[f] = o
    return tuple(results)
